# Initial kernel scaffold; baseline (speedup 1.0000x reference)
import jax
import jax.numpy as jnp
from jax.experimental import pallas as pl


def kernel(mesh_x, point_x, edge_index_m2m, edge_attr_m2m, edge_index_p2m, edge_attr_p2m, global_features, params):
    raise NotImplementedError("write your pallas kernel here")


# speedup vs baseline: 2.2965x; 2.2965x over previous
import functools

import jax
import jax.numpy as jnp
from jax import lax
from jax.experimental import pallas as pl
from jax.experimental.pallas import tpu as pltpu
from jax.experimental.pallas import tpu_sc as plsc

F32 = jnp.float32
_NC, _NS = 2, 16
_NW = _NC * _NS
_IDXW = 128
_ROWS_I = 5
_CHUNK_E = _IDXW * _ROWS_I


def _leaky(x):
    return jnp.where(x >= 0, x, x * F32(0.01))


def _dot(a, b):
    return jnp.dot(a, b, preferred_element_type=F32)


def _full(shape):
    return pl.BlockSpec(shape, lambda i: (0,) * len(shape))


def _row(shape):
    return pl.BlockSpec(shape, lambda i: (i,) + (0,) * (len(shape) - 1))


def _enc_edges(attr_m, attr_p, wm, bm, wp, bp):
    E = attr_m.shape[0]
    R = 2000

    def body(am, ap, wmr, bmr, wpr, bpr, om, op):
        om[...] = _leaky(_dot(am[...], wmr[...]) + bmr[...])
        op[...] = _leaky(_dot(ap[...], wpr[...]) + bpr[...])

    return pl.pallas_call(
        body, grid=(E // R,),
        in_specs=[_row((R, 16)), _row((R, 16)), _full((16, 32)),
                  _full((1, 32)), _full((16, 32)), _full((1, 32))],
        out_specs=[_row((R, 32)), _row((R, 32))],
        out_shape=[jax.ShapeDtypeStruct((E, 32), F32)] * 2,
    )(attr_m, attr_p, wm, bm, wp, bp)


def _enc_nodes_pre(mesh_x, point_x, w_em, b_em, w_ep, b_ep, a_sm, a_dm, a_dp, a_sp):
    N = mesh_x.shape[0]
    R = 1000

    def body(mx, px, wm, bm, wp, bp, t_sm, t_dm, t_dp, t_sp,
             hm_o, hp_o, sm_o, dm_o, dp_o, sp_o):
        hm = _leaky(_dot(mx[...], wm[...]) + bm[...])
        hp = _leaky(_dot(px[...], wp[...]) + bp[...])
        hm_o[...] = hm
        hp_o[...] = hp
        sm_o[...] = _dot(hm, t_sm[...])
        dm_o[...] = _dot(hm, t_dm[...])
        dp_o[...] = _dot(hm, t_dp[...])
        sp_o[...] = _dot(hp, t_sp[...])

    return pl.pallas_call(
        body, grid=(N // R,),
        in_specs=[_row((R, 128)), _row((R, 128)), _full((128, 32)),
                  _full((1, 32)), _full((128, 32)), _full((1, 32))] +
                 [_full((32, 32))] * 4,
        out_specs=[_row((R, 32))] * 6,
        out_shape=[jax.ShapeDtypeStruct((N, 32), F32)] * 6,
    )(mesh_x, point_x, w_em, b_em, w_ep, b_ep, a_sm, a_dm, a_dp, a_sp)


def _edge_update(he, gs, gd, c, w1, w2, b2):
    E = he.shape[0]
    R = 2000

    def body(her, gsr, gdr, cr, w1r, w2r, b2r, out, ssum):
        he_v = her[...]
        t = _leaky(_dot(he_v, w1r[...]) + gsr[...] + gdr[...] + cr[...])
        o = he_v + _dot(t, w2r[...]) + b2r[...]
        out[...] = o

        @pl.when(pl.program_id(0) == 0)
        def _():
            ssum[...] = jnp.zeros_like(ssum)
        ssum[...] += jnp.sum(o, axis=0, keepdims=True)

    return pl.pallas_call(
        body, grid=(E // R,),
        in_specs=[_row((R, 32))] * 3 + [_full((1, 32)), _full((32, 32)),
                                        _full((32, 32)), _full((1, 32))],
        out_specs=[_row((R, 32)), _full((1, 32))],
        out_shape=[jax.ShapeDtypeStruct((E, 32), F32),
                   jax.ShapeDtypeStruct((1, 32), F32)],
    )(he, gs, gd, c, w1, w2, b2)


def _node_update(hm, agg2, hp, c_n, c_p, ws, extra, last):
    N = hm.shape[0]
    R = 1000
    G = N // R
    a1, a2, a3, w2n, b2n, p1, w2p, b2p = ws

    def body(hmr, agm, agp, hpr, cnr, cpr, a1r, a2r, a3r, w2nr, b2nr,
             p1r, w2pr, b2pr, e0, e1, e2, e3, *outs):
        hm_v = hmr[...]
        x = (_dot(hm_v, a1r[...]) + _dot(agm[...], a2r[...]) +
             _dot(agp[...], a3r[...]) + cnr[...])
        hm_n = hm_v + _dot(_leaky(x), w2nr[...]) + b2nr[...]
        hp_v = hpr[...]
        hp_n = hp_v + _dot(_leaky(_dot(hp_v, p1r[...]) + cpr[...]), w2pr[...]) + b2pr[...]
        outs[0][...] = hm_n
        outs[1][...] = hp_n

        @pl.when(pl.program_id(0) == 0)
        def _():
            outs[2][...] = jnp.zeros_like(outs[2])
            outs[3][...] = jnp.zeros_like(outs[3])
        outs[2][...] += jnp.sum(hm_n, axis=0, keepdims=True)
        outs[3][...] += jnp.sum(hp_n, axis=0, keepdims=True)
        if last:
            d = _leaky(_dot(hm_n, e0[...]) + e1[...])
            outs[4][...] = _dot(d, e2[...]) + e3[...]
        else:
            outs[4][...] = _dot(hm_n, e0[...])
            outs[5][...] = _dot(hm_n, e1[...])
            outs[6][...] = _dot(hm_n, e2[...])
            outs[7][...] = _dot(hp_n, e3[...])

    if last:
        extra_specs = [_full((32, 32)), _full((1, 32)), _full((32, 3)), _full((1, 3))]
        out_specs = [_row((R, 32))] * 2 + [_full((1, 32))] * 2 + [_row((R, 3))]
        out_shape = ([jax.ShapeDtypeStruct((N, 32), F32)] * 2 +
                     [jax.ShapeDtypeStruct((1, 32), F32)] * 2 +
                     [jax.ShapeDtypeStruct((N, 3), F32)])
    else:
        extra_specs = [_full((32, 32))] * 4
        out_specs = [_row((R, 32))] * 2 + [_full((1, 32))] * 2 + [_row((R, 32))] * 4
        out_shape = ([jax.ShapeDtypeStruct((N, 32), F32)] * 2 +
                     [jax.ShapeDtypeStruct((1, 32), F32)] * 2 +
                     [jax.ShapeDtypeStruct((N, 32), F32)] * 4)

    return pl.pallas_call(
        body, grid=(G,),
        in_specs=[_row((R, 32)),
                  pl.BlockSpec((R, 32), lambda i: (i, 0)),
                  pl.BlockSpec((R, 32), lambda i: (i + G, 0)),
                  _row((R, 32)), _full((1, 32)), _full((1, 32))] +
                 [_full((32, 32))] * 4 + [_full((1, 32))] +
                 [_full((32, 32))] * 2 + [_full((1, 32))] + extra_specs,
        out_specs=out_specs, out_shape=out_shape,
    )(hm, agg2, agg2, hp, c_n, c_p, *ws[:5], *ws[5:], *extra)


def _global_init(gf, w_eg, b_eg, cws, cbs):
    def body(gfr, wr, br, cw0, cw1, cw2, cw3, cb0, cb1, cb2, cb3,
             hg_o, c0, c1, c2, c3):
        hg = _leaky(_dot(gfr[...], wr[...]) + br[...])
        hg_o[...] = hg
        c0[...] = _dot(hg, cw0[...]) + cb0[...]
        c1[...] = _dot(hg, cw1[...]) + cb1[...]
        c2[...] = _dot(hg, cw2[...]) + cb2[...]
        c3[...] = _dot(hg, cw3[...]) + cb3[...]

    return pl.pallas_call(
        body, grid=(1,),
        in_specs=[_full((1, 8)), _full((8, 32)), _full((1, 32))] +
                 [_full((32, 32))] * 4 + [_full((1, 32))] * 4,
        out_specs=[_full((1, 32))] * 5,
        out_shape=[jax.ShapeDtypeStruct((1, 32), F32)] * 5,
    )(gf, w_eg, b_eg, *cws, *cbs)


def _global_update(hg, s_hm, s_hp, s_hem, s_hep, g1s, b1, w2, b2,
                   cws, cbs, n_nodes, n_edges, last):
    inv_n = 1.0 / n_nodes
    inv_e = 1.0 / n_edges

    def body(hgr, shm, shp, shem, shep, g0, g1, g2, g3, g4, b1r, w2r, b2r,
             *rest):
        if last:
            outs = rest
        else:
            cw0, cw1, cw2, cw3, cb0, cb1, cb2, cb3 = rest[:8]
            outs = rest[8:]
        hg_v = hgr[...]
        x = (_dot(hg_v, g0[...]) + _dot(shm[...] * inv_n, g1[...]) +
             _dot(shp[...] * inv_n, g2[...]) + _dot(shem[...] * inv_e, g3[...]) +
             _dot(shep[...] * inv_e, g4[...]) + b1r[...])
        hg_n = hg_v + _dot(_leaky(x), w2r[...]) + b2r[...]
        outs[0][...] = hg_n
        if not last:
            outs[1][...] = _dot(hg_n, cw0[...]) + cb0[...]
            outs[2][...] = _dot(hg_n, cw1[...]) + cb1[...]
            outs[3][...] = _dot(hg_n, cw2[...]) + cb2[...]
            outs[4][...] = _dot(hg_n, cw3[...]) + cb3[...]

    n_out = 1 if last else 5
    extra_in = [] if last else list(cws) + list(cbs)
    extra_specs = [] if last else [_full((32, 32))] * 4 + [_full((1, 32))] * 4
    return pl.pallas_call(
        body, grid=(1,),
        in_specs=[_full((1, 32))] * 5 + [_full((32, 32))] * 5 +
                 [_full((1, 32))] + [_full((32, 32))] + [_full((1, 32))] +
                 extra_specs,
        out_specs=[_full((1, 32))] * n_out,
        out_shape=[jax.ShapeDtypeStruct((1, 32), F32)] * n_out,
    )(hg, s_hm, s_hp, s_hem, s_hep, *g1s, b1, w2, b2, *extra_in)


def _sc_gather(t_sm, i_sm, t_dm, i_dm, t_sp, i_sp, t_dp, i_dp, n_edges):
    nchunk = n_edges // _CHUNK_E
    pairs = (nchunk + 2 * _NW - 1) // (2 * _NW)
    mesh = plsc.VectorSubcoreMesh(core_axis_name="c", subcore_axis_name="s", num_cores=_NC, num_subcores=_NS)

    @functools.partial(
        pl.kernel, mesh=mesh,
        compiler_params=pltpu.CompilerParams(use_tc_tiling_on_sc=False),
        out_type=[jax.ShapeDtypeStruct((n_edges, 32), F32)] * 4,
        scratch_types=[pltpu.VMEM((2, _ROWS_I, _IDXW), jnp.int32),
                       pltpu.VMEM((2 * _CHUNK_E, 32), F32),
                       pltpu.SemaphoreType.DMA, pltpu.SemaphoreType.DMA],
    )
    def k(tsm, ism, tdm, idm, tsp, isp, tdp, idp, o0, o1, o2, o3, ib, rv, s0, s1):
        w = lax.axis_index("s") * _NC + lax.axis_index("c")
        sems = (s0, s1)
        for tab, idx, out in ((tsm, ism, o0), (tdm, idm, o1),
                              (tsp, isp, o2), (tdp, idp, o3)):
            def start_chunk(ch, b, tab=tab, idx=idx):
                pltpu.sync_copy(idx.at[ch], ib.at[b])
                for r in range(_ROWS_I):
                    pltpu.async_copy(
                        tab.at[ib.at[b, r]],
                        rv.at[pl.ds(b * _CHUNK_E + r * _IDXW, _IDXW), :],
                        sems[b])

            def drain_chunk(ch, b, tab=tab, out=out):
                for r in range(_ROWS_I):
                    pltpu.make_async_copy(
                        tab.at[ib.at[b, r]],
                        rv.at[pl.ds(b * _CHUNK_E + r * _IDXW, _IDXW), :],
                        sems[b]).wait()
                pltpu.sync_copy(rv.at[pl.ds(b * _CHUNK_E, _CHUNK_E), :],
                                out.at[pl.ds(ch * _CHUNK_E, _CHUNK_E), :])

            def pair_body(j, carry):
                ch0 = w + _NW * (2 * j)
                ch1 = ch0 + _NW

                @pl.when(ch0 < nchunk)
                def _():
                    start_chunk(ch0, 0)

                @pl.when(ch1 < nchunk)
                def _():
                    start_chunk(ch1, 1)

                @pl.when(ch0 < nchunk)
                def _():
                    drain_chunk(ch0, 0)

                @pl.when(ch1 < nchunk)
                def _():
                    drain_chunk(ch1, 1)
                return carry

            lax.fori_loop(0, pairs, pair_body, 0)

    return k(t_sm, i_sm, t_dm, i_dm, t_sp, i_sp, t_dp, i_dp)


def _sc_scatter(vals_m, idx_m, vals_p, idx_p, zeros, n_nodes, n_edges):
    nchunk = n_edges // _CHUNK_E
    jmax = (nchunk + _NS - 1) // _NS
    grp = n_nodes // 8
    big = grp - (grp // _NS) * _NS
    r_big, r_sml = (grp // _NS + 1) * 8, (grp // _NS) * 8
    mesh = plsc.VectorSubcoreMesh(core_axis_name="c", subcore_axis_name="s", num_cores=_NC, num_subcores=_NS)

    @functools.partial(
        pl.kernel, mesh=mesh,
        compiler_params=pltpu.CompilerParams(use_tc_tiling_on_sc=False),
        out_type=jax.ShapeDtypeStruct((2 * n_nodes, 32), F32),
        scratch_types=[pltpu.VMEM_SHARED((n_nodes, 32), F32),
                       pltpu.VMEM((_ROWS_I, _IDXW), jnp.int32),
                       pltpu.VMEM((_CHUNK_E, 32), F32)],
    )
    def k(vm, im, vp, ip, zz, out, acc, iv, vv):
        c = lax.axis_index("c")
        s = lax.axis_index("s")
        base = jnp.minimum(s, big) * r_big + jnp.maximum(s - big, 0) * r_sml

        def own_slab(fn):
            @pl.when(s < big)
            def _():
                fn(r_big)

            @pl.when(s >= big)
            def _():
                fn(r_sml)

        own_slab(lambda rows: pltpu.sync_copy(
            zz.at[pl.ds(base, rows), :], acc.at[pl.ds(base, rows), :]))
        plsc.subcore_barrier()

        def run(vals, idx):
            def body(j, carry):
                ch = s + _NS * j

                @pl.when(ch < nchunk)
                def _():
                    pltpu.sync_copy(idx.at[ch], iv)
                    pltpu.sync_copy(vals.at[pl.ds(ch * _CHUNK_E, _CHUNK_E), :], vv)
                    for r in range(_ROWS_I):
                        pltpu.sync_copy(vv.at[pl.ds(r * _IDXW, _IDXW), :],
                                        acc.at[iv.at[r]], add=True)
                return carry
            lax.fori_loop(0, jmax, body, 0)

        @pl.when(c == 0)
        def _():
            run(vm, im)

        @pl.when(c == 1)
        def _():
            run(vp, ip)

        plsc.subcore_barrier()
        own_slab(lambda rows: pltpu.sync_copy(
            acc.at[pl.ds(base, rows), :],
            out.at[pl.ds(c * n_nodes + base, rows), :]))

    return k(vals_m, idx_m, vals_p, idx_p, zeros)


def kernel(mesh_x, point_x, edge_index_m2m, edge_attr_m2m, edge_index_p2m,
           edge_attr_p2m, global_features, params):
    N = mesh_x.shape[0]
    E = edge_attr_m2m.shape[0]
    p = params
    blocks = p["blocks"]
    nb = len(blocks)

    def b2(v):
        return v.reshape(1, -1)

    def edge_w(blk, name):
        W1 = blk[name]["l1"]["W"]
        return (W1[0:32], W1[32:64], W1[64:96], W1[96:128],
                b2(blk[name]["l1"]["b"]), blk[name]["l2"]["W"],
                b2(blk[name]["l2"]["b"]))

    em = [edge_w(blk, "edge_m2m") for blk in blocks]
    ep = [edge_w(blk, "edge_p2m") for blk in blocks]
    nm = [(blk["node_mesh"]["l1"]["W"][0:32], blk["node_mesh"]["l1"]["W"][32:64],
           blk["node_mesh"]["l1"]["W"][64:96], blk["node_mesh"]["l1"]["W"][96:128],
           b2(blk["node_mesh"]["l1"]["b"]), blk["node_mesh"]["l2"]["W"],
           b2(blk["node_mesh"]["l2"]["b"])) for blk in blocks]
    npt = [(blk["node_point"]["l1"]["W"][0:32], blk["node_point"]["l1"]["W"][32:64],
            b2(blk["node_point"]["l1"]["b"]), blk["node_point"]["l2"]["W"],
            b2(blk["node_point"]["l2"]["b"])) for blk in blocks]
    gl = [([blk["global"]["l1"]["W"][k * 32:(k + 1) * 32] for k in range(5)],
           b2(blk["global"]["l1"]["b"]), blk["global"]["l2"]["W"],
           b2(blk["global"]["l2"]["b"])) for blk in blocks]

    def const_ws(i):
        cws = (em[i][3], ep[i][3], nm[i][3], npt[i][1])
        cbs = (em[i][4], ep[i][4], nm[i][4], npt[i][2])
        return cws, cbs

    nch = E // _CHUNK_E
    i_sm = edge_index_m2m[0].reshape(nch, _ROWS_I, _IDXW)
    i_dm = edge_index_m2m[1].reshape(nch, _ROWS_I, _IDXW)
    i_sp = edge_index_p2m[0].reshape(nch, _ROWS_I, _IDXW)
    i_dp = edge_index_p2m[1].reshape(nch, _ROWS_I, _IDXW)

    zeros = jnp.zeros((N, 32), F32)

    he_m, he_p = _enc_edges(edge_attr_m2m, edge_attr_p2m,
                            p["enc_e_m2m"]["W"], b2(p["enc_e_m2m"]["b"]),
                            p["enc_e_p2m"]["W"], b2(p["enc_e_p2m"]["b"]))
    hm, hp, t_sm, t_dm, t_dp, t_sp = _enc_nodes_pre(
        mesh_x, point_x, p["enc_mesh"]["W"], b2(p["enc_mesh"]["b"]),
        p["enc_point"]["W"], b2(p["enc_point"]["b"]),
        em[0][1], em[0][2], ep[0][2], ep[0][1])
    cws0, cbs0 = const_ws(0)
    hg, c_em, c_ep, c_nm, c_np = _global_init(
        global_features, p["enc_g"]["W"], b2(p["enc_g"]["b"]), cws0, cbs0)

    dec = None
    for i in range(nb):
        last = i == nb - 1
        gs_m, gd_m, gs_p, gd_p = _sc_gather(
            t_sm, i_sm, t_dm, i_dm, t_sp, i_sp, t_dp, i_dp, E)
        he_m, s_hem = _edge_update(he_m, gs_m, gd_m, c_em,
                                   em[i][0], em[i][5], em[i][6])
        he_p, s_hep = _edge_update(he_p, gs_p, gd_p, c_ep,
                                   ep[i][0], ep[i][5], ep[i][6])
        agg2 = _sc_scatter(he_m, i_dm, he_p, i_dp, zeros, N, E)
        ws = (nm[i][0], nm[i][1], nm[i][2], nm[i][5], nm[i][6],
              npt[i][0], npt[i][3], npt[i][4])
        if last:
            extra = (p["dec1"]["W"], b2(p["dec1"]["b"]),
                     p["dec2"]["W"], b2(p["dec2"]["b"]))
            hm, hp, s_hm, s_hp, dec = _node_update(
                hm, agg2, hp, c_nm, c_np, ws, extra, True)
            hg, = _global_update(hg, s_hm, s_hp, s_hem, s_hep,
                                 gl[i][0], gl[i][1], gl[i][2], gl[i][3],
                                 None, None, N, E, True)
        else:
            extra = (em[i + 1][1], em[i + 1][2], ep[i + 1][2], ep[i + 1][1])
            hm, hp, s_hm, s_hp, t_sm, t_dm, t_dp, t_sp = _node_update(
                hm, agg2, hp, c_nm, c_np, ws, extra, False)
            cws, cbs = const_ws(i + 1)
            hg, c_em, c_ep, c_nm, c_np = _global_update(
                hg, s_hm, s_hp, s_hem, s_hep,
                gl[i][0], gl[i][1], gl[i][2], gl[i][3], cws, cbs, N, E, False)

    return (dec, he_m, he_p, hg)
